# baseline (speedup 1.0000x reference)
import functools

import jax
import jax.numpy as jnp
from jax import lax
from jax.experimental import pallas as pl
from jax.experimental.pallas import tpu as pltpu
from jax.experimental.pallas import tpu_sc as plsc

N = 100000
D_X = 128
D_E = 12
D_OUT = D_E + D_X

NUM_CORES = 2
NUM_SUBCORES = 16
NW = NUM_CORES * NUM_SUBCORES

C = 400
PER_W = 3200
TAIL_W = N - (NW - 1) * PER_W
NK_FULL = PER_W // C
NK_TAIL = TAIL_W // C


def _sc_concat_embed(residue, x, table):
    mesh = plsc.VectorSubcoreMesh(core_axis_name="c", subcore_axis_name="s")

    @functools.partial(
        pl.kernel,
        mesh=mesh,
        out_type=jax.ShapeDtypeStruct((N, D_OUT), jnp.float32),
        scratch_types=[
            pltpu.VMEM((C,), jnp.int32),
            pltpu.VMEM((20, D_E), jnp.float32),
            pltpu.VMEM((C, D_X), jnp.float32),
            pltpu.VMEM((C, D_OUT), jnp.float32),
        ],
        compiler_params=pltpu.CompilerParams(
            use_tc_tiling_on_sc=False, needs_layout_passes=False
        ),
    )
    def k(res_hbm, x_hbm, tab_hbm, out_hbm, idx_v, tab_v, xb_v, stage_v):
        wid = lax.axis_index("s") * NUM_CORES + lax.axis_index("c")
        pltpu.sync_copy(tab_hbm, tab_v)

        lanes = lax.iota(jnp.int32, 16)

        def chunk(j, base_w):
            base = base_w + j * C
            pltpu.sync_copy(res_hbm.at[pl.ds(base, C)], idx_v)
            pltpu.sync_copy(x_hbm.at[pl.ds(base, C), :], xb_v)

            def emb_group(i, carry):
                idx16 = idx_v[pl.ds(i * 16, 16)]
                row_ids = i * 16 + lanes
                for c in range(D_E):
                    csplat = jnp.full((16,), c, jnp.int32)
                    vals = plsc.load_gather(tab_v, [idx16, csplat])
                    plsc.store_scatter(stage_v, [row_ids, csplat], vals)
                return carry

            lax.fori_loop(0, C // 16, emb_group, 0)

            def x_row(r, carry):
                rsplat = jnp.full((16,), r, jnp.int32)
                for s in range(D_X // 16):
                    vals = xb_v[r, pl.ds(s * 16, 16)]
                    cols = D_E + s * 16 + lanes
                    plsc.store_scatter(stage_v, [rsplat, cols], vals)
                return carry

            lax.fori_loop(0, C, x_row, 0)

            pltpu.sync_copy(stage_v, out_hbm.at[pl.ds(base, C), :])
            return base_w

        nk = jnp.where(wid == NW - 1, NK_TAIL, NK_FULL)
        lax.fori_loop(0, nk, chunk, wid * PER_W)

    return k(residue, x, table)


def kernel(residue, x, embed_weight):
    return _sc_concat_embed(residue, x, embed_weight)


# speedup vs baseline: 3.0657x; 3.0657x over previous
import functools

import jax
import jax.numpy as jnp
from jax import lax
from jax.experimental import pallas as pl
from jax.experimental.pallas import tpu as pltpu
from jax.experimental.pallas import tpu_sc as plsc

N = 100000
D_X = 128
D_E = 12
D_OUT = D_E + D_X

NUM_CORES = 2
NUM_SUBCORES = 16
NW = NUM_CORES * NUM_SUBCORES

PER_W = 3120
TAIL_W = N - (NW - 1) * PER_W

TC_BLOCK = 10000


def _sc_gather(residue, table):
    mesh = plsc.VectorSubcoreMesh(core_axis_name="c", subcore_axis_name="s")

    @functools.partial(
        pl.kernel,
        mesh=mesh,
        out_type=jax.ShapeDtypeStruct((N, D_E), jnp.float32),
        scratch_types=[
            pltpu.VMEM((TAIL_W,), jnp.int32),
            pltpu.VMEM((20, D_E), jnp.float32),
            pltpu.VMEM((TAIL_W, D_E), jnp.float32),
        ],
        compiler_params=pltpu.CompilerParams(
            use_tc_tiling_on_sc=False, needs_layout_passes=False
        ),
    )
    def k(res_hbm, tab_hbm, out_hbm, idx_v, tab_v, rows_v):
        wid = lax.axis_index("s") * NUM_CORES + lax.axis_index("c")
        pltpu.sync_copy(tab_hbm, tab_v)

        lanes = lax.iota(jnp.int32, 16)

        def run(base, z, g):
            pltpu.sync_copy(res_hbm.at[pl.ds(base, z)], idx_v.at[pl.ds(0, z)])

            def group(i, carry):
                idx16 = idx_v[pl.ds(i * 16, 16)]
                row_ids = i * 16 + lanes
                for c in range(D_E):
                    csplat = jnp.full((16,), c, jnp.int32)
                    vals = plsc.load_gather(tab_v, [idx16, csplat])
                    plsc.store_scatter(rows_v, [row_ids, csplat], vals)
                return carry

            lax.fori_loop(0, g, group, 0)

            pltpu.sync_copy(
                rows_v.at[pl.ds(0, z), :],
                out_hbm.at[pl.ds(base, z), :],
            )

        @pl.when(wid < NW - 1)
        def _():
            run(wid * PER_W, PER_W, PER_W // 16)

        @pl.when(wid == NW - 1)
        def _():
            run((NW - 1) * PER_W, TAIL_W, TAIL_W // 16)

    return k(residue, table)


def _tc_concat(emb, x):
    grid = (N // TC_BLOCK,)
    s_e = jnp.concatenate(
        [jnp.eye(D_E, dtype=jnp.float32),
         jnp.zeros((D_E, D_X), jnp.float32)], axis=1)
    s_x = jnp.concatenate(
        [jnp.zeros((D_X, D_E), jnp.float32),
         jnp.eye(D_X, dtype=jnp.float32)], axis=1)

    def body(emb_ref, x_ref, se_ref, sx_ref, o_ref):
        o_ref[...] = jax.lax.dot(
            emb_ref[...], se_ref[...],
            preferred_element_type=jnp.float32,
        ) + jax.lax.dot(
            x_ref[...], sx_ref[...],
            preferred_element_type=jnp.float32,
        )

    return pl.pallas_call(
        body,
        grid=grid,
        in_specs=[
            pl.BlockSpec((TC_BLOCK, D_E), lambda i: (i, 0)),
            pl.BlockSpec((TC_BLOCK, D_X), lambda i: (i, 0)),
            pl.BlockSpec((D_E, D_OUT), lambda i: (0, 0)),
            pl.BlockSpec((D_X, D_OUT), lambda i: (0, 0)),
        ],
        out_specs=pl.BlockSpec((TC_BLOCK, D_OUT), lambda i: (i, 0)),
        out_shape=jax.ShapeDtypeStruct((N, D_OUT), jnp.float32),
    )(emb, x, s_e, s_x)


def kernel(residue, x, embed_weight):
    emb = _sc_gather(residue, embed_weight)
    return _tc_concat(emb, x)


# speedup vs baseline: 3.0834x; 1.0058x over previous
import functools

import jax
import jax.numpy as jnp
from jax import lax
from jax.experimental import pallas as pl
from jax.experimental.pallas import tpu as pltpu
from jax.experimental.pallas import tpu_sc as plsc

N = 100000
D_X = 128
D_E = 12
D_OUT = D_E + D_X

NUM_CORES = 2
NUM_SUBCORES = 16
NW = NUM_CORES * NUM_SUBCORES

PER_W = 3120
TAIL_W = N - (NW - 1) * PER_W

TC_BLOCK = 4000


def _sc_gather(residue, table):
    mesh = plsc.VectorSubcoreMesh(core_axis_name="c", subcore_axis_name="s")

    @functools.partial(
        pl.kernel,
        mesh=mesh,
        out_type=jax.ShapeDtypeStruct((N, D_E), jnp.float32),
        scratch_types=[
            pltpu.VMEM((TAIL_W,), jnp.int32),
            pltpu.VMEM((20, D_E), jnp.float32),
            pltpu.VMEM((TAIL_W, D_E), jnp.float32),
        ],
        compiler_params=pltpu.CompilerParams(
            use_tc_tiling_on_sc=False, needs_layout_passes=False
        ),
    )
    def k(res_hbm, tab_hbm, out_hbm, idx_v, tab_v, rows_v):
        wid = lax.axis_index("s") * NUM_CORES + lax.axis_index("c")
        pltpu.sync_copy(tab_hbm, tab_v)

        lanes = lax.iota(jnp.int32, 16)

        def run(base, z, g):
            pltpu.sync_copy(res_hbm.at[pl.ds(base, z)], idx_v.at[pl.ds(0, z)])

            def group(i, carry):
                idx16 = idx_v[pl.ds(i * 16, 16)]
                row_ids = i * 16 + lanes
                for c in range(D_E):
                    csplat = jnp.full((16,), c, jnp.int32)
                    vals = plsc.load_gather(tab_v, [idx16, csplat])
                    plsc.store_scatter(rows_v, [row_ids, csplat], vals)
                return carry

            lax.fori_loop(0, g, group, 0)

            pltpu.sync_copy(
                rows_v.at[pl.ds(0, z), :],
                out_hbm.at[pl.ds(base, z), :],
            )

        @pl.when(wid < NW - 1)
        def _():
            run(wid * PER_W, PER_W, PER_W // 16)

        @pl.when(wid == NW - 1)
        def _():
            run((NW - 1) * PER_W, TAIL_W, TAIL_W // 16)

    return k(residue, table)


def _tc_concat(emb, x):
    grid = (N // TC_BLOCK,)

    def body(emb_ref, x_ref, o_ref):
        o_ref[...] = jnp.concatenate([emb_ref[...], x_ref[...]], axis=1)

    return pl.pallas_call(
        body,
        grid=grid,
        in_specs=[
            pl.BlockSpec((TC_BLOCK, D_E), lambda i: (i, 0)),
            pl.BlockSpec((TC_BLOCK, D_X), lambda i: (i, 0)),
        ],
        out_specs=pl.BlockSpec((TC_BLOCK, D_OUT), lambda i: (i, 0)),
        out_shape=jax.ShapeDtypeStruct((N, D_OUT), jnp.float32),
    )(emb, x)


def kernel(residue, x, embed_weight):
    emb = _sc_gather(residue, embed_weight)
    return _tc_concat(emb, x)


# speedup vs baseline: 3.1424x; 1.0191x over previous
import functools

import jax
import jax.numpy as jnp
from jax import lax
from jax.experimental import pallas as pl
from jax.experimental.pallas import tpu as pltpu
from jax.experimental.pallas import tpu_sc as plsc

N = 100000
H = N // 2
D_X = 128
D_E = 12
D_OUT = D_E + D_X

NUM_CORES = 2
NUM_SUBCORES = 16
NW = NUM_CORES * NUM_SUBCORES

PER_W = (H // NW) // 16 * 16
TAIL_W = H - (NW - 1) * PER_W

TC_BLOCK = 10000
HB = H // TC_BLOCK


def _sc_gather(residue_h, table):
    mesh = plsc.VectorSubcoreMesh(core_axis_name="c", subcore_axis_name="s")

    @functools.partial(
        pl.kernel,
        mesh=mesh,
        out_type=jax.ShapeDtypeStruct((H, D_E), jnp.float32),
        scratch_types=[
            pltpu.VMEM((TAIL_W,), jnp.int32),
            pltpu.VMEM((20, D_E), jnp.float32),
            pltpu.VMEM((TAIL_W, D_E), jnp.float32),
        ],
        compiler_params=pltpu.CompilerParams(
            use_tc_tiling_on_sc=False, needs_layout_passes=False
        ),
    )
    def k(res_hbm, tab_hbm, out_hbm, idx_v, tab_v, rows_v):
        wid = lax.axis_index("s") * NUM_CORES + lax.axis_index("c")
        pltpu.sync_copy(tab_hbm, tab_v)

        lanes = lax.iota(jnp.int32, 16)

        def run(base, z, g):
            pltpu.sync_copy(res_hbm.at[pl.ds(base, z)], idx_v.at[pl.ds(0, z)])

            def group(i, carry):
                idx16 = idx_v[pl.ds(i * 16, 16)]
                row_ids = i * 16 + lanes
                for c in range(D_E):
                    csplat = jnp.full((16,), c, jnp.int32)
                    vals = plsc.load_gather(tab_v, [idx16, csplat])
                    plsc.store_scatter(rows_v, [row_ids, csplat], vals)
                return carry

            lax.fori_loop(0, g, group, 0)

            pltpu.sync_copy(
                rows_v.at[pl.ds(0, z), :],
                out_hbm.at[pl.ds(base, z), :],
            )

        @pl.when(wid < NW - 1)
        def _():
            run(wid * PER_W, PER_W, PER_W // 16)

        @pl.when(wid == NW - 1)
        def _():
            run((NW - 1) * PER_W, TAIL_W, TAIL_W // 16)

    return k(residue_h, table)


def _tc_concat_half(emb_h, x, half, prev=None):
    def body(emb_ref, x_ref, *rest):
        o_ref = rest[-1]
        o_ref[...] = jnp.concatenate([emb_ref[...], x_ref[...]], axis=1)

    in_specs = [
        pl.BlockSpec((TC_BLOCK, D_E), lambda i: (i, 0)),
        pl.BlockSpec((TC_BLOCK, D_X), lambda i, h=half: (i + HB * h, 0)),
    ]
    args = [emb_h, x]
    kwargs = {}
    if prev is not None:
        in_specs.append(pl.BlockSpec(memory_space=pl.ANY))
        args.append(prev)
        kwargs["input_output_aliases"] = {2: 0}

    return pl.pallas_call(
        body,
        grid=(HB,),
        in_specs=in_specs,
        out_specs=pl.BlockSpec((TC_BLOCK, D_OUT),
                               lambda i, h=half: (i + HB * h, 0)),
        out_shape=jax.ShapeDtypeStruct((N, D_OUT), jnp.float32),
        **kwargs,
    )(*args)


def kernel(residue, x, embed_weight):
    emb0 = _sc_gather(lax.slice(residue, (0,), (H,)), embed_weight)
    emb1 = _sc_gather(lax.slice(residue, (H,), (N,)), embed_weight)
    out = _tc_concat_half(emb0, x, 0)
    return _tc_concat_half(emb1, x, 1, prev=out)
